# baseline (speedup 1.0000x reference)
import functools

import jax
import jax.numpy as jnp
from jax import lax
from jax.experimental import pallas as pl
from jax.experimental.pallas import tpu as pltpu
from jax.experimental.pallas import tpu_sc as plsc

_GAMMA = 12.0
_D = 64
_B = 16384
_NC = 2
_NS = 16
_NW = _NC * _NS
_BPW = _B // _NW
_NCHUNK = 4
_CROWS = _BPW // _NCHUNK
_CGROUPS = _CROWS // 16


def _tec_body(hidx_hbm, hoff_hbm, tidx_hbm, toff_hbm, ridx_hbm, roff_hbm,
              ent_hbm, rel_hbm, out_hbm,
              hidx_v, hoff_v, tidx_v, toff_v, ridx_v, roff_v,
              h_v, t_v, r_v, out_v, sem0, sem1):
    c = lax.axis_index("c")
    s = lax.axis_index("s")
    wid = s * _NC + c
    base = wid * _BPW

    for src, dst in ((hidx_hbm, hidx_v), (hoff_hbm, hoff_v),
                     (tidx_hbm, tidx_v), (toff_hbm, toff_v),
                     (ridx_hbm, ridx_v), (roff_hbm, roff_v)):
        pltpu.sync_copy(src.at[pl.ds(base, _BPW)], dst)

    bufs = (h_v, t_v, r_v)
    sems = (sem0, sem1)

    def fire(ch):
        off = ch * _CROWS
        sl = pl.ds(off, _CROWS)
        buf = ch % 2
        return (
            pltpu.async_copy(ent_hbm.at[hidx_v.at[sl]], h_v.at[buf], sems[buf]),
            pltpu.async_copy(ent_hbm.at[tidx_v.at[sl]], t_v.at[buf], sems[buf]),
            pltpu.async_copy(rel_hbm.at[ridx_v.at[sl]], r_v.at[buf], sems[buf]),
        )

    lane = lax.iota(jnp.int32, 16)

    def make_group(buf):
        def group(g, carry):
            rowbase = g * 16
            brow = rowbase - (rowbase // _CROWS) * _CROWS
            phv = hoff_v[pl.ds(rowbase, 16)]
            ptv = toff_v[pl.ds(rowbase, 16)]
            prv = roff_v[pl.ds(rowbase, 16)]
            vec = jnp.zeros((16,), jnp.float32)
            for i in range(16):
                br = brow + i
                ph = phv[i]
                pt = ptv[i]
                pr = prv[i]
                acc = jnp.zeros((16,), jnp.float32)
                for cchunk in range(_D // 16):
                    hv = h_v[buf, br, pl.ds(ph + cchunk * 16, 16)]
                    tv = t_v[buf, br, pl.ds(pt + cchunk * 16, 16)]
                    rv = r_v[buf, br, pl.ds(pr + cchunk * 16, 16)]
                    acc = acc + jnp.abs(hv + rv - tv)
                vec = jnp.where(lane == i, jnp.sum(acc), vec)
            out_v[pl.ds(rowbase, 16)] = _GAMMA - vec
            return carry
        return group

    copies = {0: fire(0)}
    for ch in range(_NCHUNK):
        for cp in copies.pop(ch):
            cp.wait()
        if ch + 1 < _NCHUNK:
            copies[ch + 1] = fire(ch + 1)
        lax.fori_loop(ch * _CGROUPS, (ch + 1) * _CGROUPS,
                      make_group(ch % 2), 0)

    pltpu.sync_copy(out_v, out_hbm.at[pl.ds(base, _BPW)])


@functools.partial(
    pl.kernel,
    out_type=jax.ShapeDtypeStruct((_B,), jnp.float32),
    mesh=plsc.VectorSubcoreMesh(core_axis_name="c", subcore_axis_name="s"),
    compiler_params=pltpu.CompilerParams(
        needs_layout_passes=False, use_tc_tiling_on_sc=True),
    scratch_types=[
        pltpu.VMEM((_BPW,), jnp.int32),
        pltpu.VMEM((_BPW,), jnp.int32),
        pltpu.VMEM((_BPW,), jnp.int32),
        pltpu.VMEM((_BPW,), jnp.int32),
        pltpu.VMEM((_BPW,), jnp.int32),
        pltpu.VMEM((_BPW,), jnp.int32),
        pltpu.VMEM((2, _CROWS, 2 * _D), jnp.float32),
        pltpu.VMEM((2, _CROWS, 2 * _D), jnp.float32),
        pltpu.VMEM((2, _CROWS, 2 * _D), jnp.float32),
        pltpu.VMEM((_BPW,), jnp.float32),
        pltpu.SemaphoreType.DMA,
        pltpu.SemaphoreType.DMA,
    ],
)
def _transe_sc(*refs):
    _tec_body(*refs)


def kernel(indices, relations, entity_embedding, relation_embedding):
    hidx = indices[:, 0].astype(jnp.int32)
    tidx = indices[:, 1].astype(jnp.int32)
    ridx = relations.astype(jnp.int32)
    ent2 = entity_embedding.reshape(entity_embedding.shape[0] // 2, 2 * _D)
    rel2 = relation_embedding.reshape(relation_embedding.shape[0] // 2, 2 * _D)
    return _transe_sc(
        hidx >> 1, (hidx & 1) * _D,
        tidx >> 1, (tidx & 1) * _D,
        ridx >> 1, (ridx & 1) * _D,
        ent2, rel2)


# speedup vs baseline: 1.6388x; 1.6388x over previous
import functools

import jax
import jax.numpy as jnp
from jax import lax
from jax.experimental import pallas as pl
from jax.experimental.pallas import tpu as pltpu
from jax.experimental.pallas import tpu_sc as plsc

_GAMMA = 12.0
_D = 64
_B = 16384
_NC = 2
_NS = 16
_NW = _NC * _NS
_BPW = _B // _NW
_CROWS = 16
_NCHUNK = _BPW // _CROWS


def _tec_body(hidx_hbm, tidx_hbm, ridx_hbm, ent_hbm, rel_hbm, out_hbm,
              hidx_v, tidx_v, ridx_v, h_v, t_v, r_v, out_v, sem):
    c = lax.axis_index("c")
    s = lax.axis_index("s")
    wid = s * _NC + c
    base = wid * _BPW

    pltpu.sync_copy(hidx_hbm.at[pl.ds(base, _BPW)], hidx_v)
    pltpu.sync_copy(tidx_hbm.at[pl.ds(base, _BPW)], tidx_v)
    pltpu.sync_copy(ridx_hbm.at[pl.ds(base, _BPW)], ridx_v)

    lane = lax.iota(jnp.int32, 16)

    def fire(ch, bufpar):
        rowbase = ch * _CROWS
        hiv = hidx_v[pl.ds(rowbase, 16)]
        tiv = tidx_v[pl.ds(rowbase, 16)]
        riv = ridx_v[pl.ds(rowbase, 16)]
        for i in range(16):
            pltpu.async_copy(ent_hbm.at[pl.ds(hiv[i], 1), :],
                             h_v.at[bufpar, pl.ds(i, 1), :], sem)
            pltpu.async_copy(ent_hbm.at[pl.ds(tiv[i], 1), :],
                             t_v.at[bufpar, pl.ds(i, 1), :], sem)
            pltpu.async_copy(rel_hbm.at[pl.ds(riv[i], 1), :],
                             r_v.at[bufpar, pl.ds(i, 1), :], sem)

    fire(0, 0)

    def chunk_body(ch, carry):
        bufpar = lax.rem(ch, 2)
        rowbase = ch * _CROWS
        pltpu.make_async_copy(ent_hbm.at[pl.ds(0, 16), :], h_v.at[0], sem).wait()
        pltpu.make_async_copy(ent_hbm.at[pl.ds(0, 16), :], t_v.at[0], sem).wait()
        pltpu.make_async_copy(rel_hbm.at[pl.ds(0, 16), :], r_v.at[0], sem).wait()

        @pl.when(ch + 1 < _NCHUNK)
        def _prefetch():
            fire(ch + 1, lax.rem(ch + 1, 2))

        vec = jnp.zeros((16,), jnp.float32)
        for i in range(16):
            acc = jnp.zeros((16,), jnp.float32)
            for cc in range(_D // 16):
                hv = h_v[bufpar, i, pl.ds(cc * 16, 16)]
                tv = t_v[bufpar, i, pl.ds(cc * 16, 16)]
                rv = r_v[bufpar, i, pl.ds(cc * 16, 16)]
                acc = acc + jnp.abs(hv + rv - tv)
            vec = jnp.where(lane == i, jnp.sum(acc), vec)
        out_v[pl.ds(rowbase, 16)] = _GAMMA - vec
        return carry

    lax.fori_loop(0, _NCHUNK, chunk_body, 0)

    pltpu.sync_copy(out_v, out_hbm.at[pl.ds(base, _BPW)])


@functools.partial(
    pl.kernel,
    out_type=jax.ShapeDtypeStruct((_B,), jnp.float32),
    mesh=plsc.VectorSubcoreMesh(core_axis_name="c", subcore_axis_name="s"),
    compiler_params=pltpu.CompilerParams(
        needs_layout_passes=False, use_tc_tiling_on_sc=True),
    scratch_types=[
        pltpu.VMEM((_BPW,), jnp.int32),
        pltpu.VMEM((_BPW,), jnp.int32),
        pltpu.VMEM((_BPW,), jnp.int32),
        pltpu.VMEM((2, _CROWS, _D), jnp.float32),
        pltpu.VMEM((2, _CROWS, _D), jnp.float32),
        pltpu.VMEM((2, _CROWS, _D), jnp.float32),
        pltpu.VMEM((_BPW,), jnp.float32),
        pltpu.SemaphoreType.DMA,
    ],
)
def _transe_sc(*refs):
    _tec_body(*refs)


def kernel(indices, relations, entity_embedding, relation_embedding):
    hidx = indices[:, 0].astype(jnp.int32)
    tidx = indices[:, 1].astype(jnp.int32)
    ridx = relations.astype(jnp.int32)
    return _transe_sc(hidx, tidx, ridx, entity_embedding, relation_embedding)


# speedup vs baseline: 2.4051x; 1.4676x over previous
import functools

import jax
import jax.numpy as jnp
from jax import lax
from jax.experimental import pallas as pl
from jax.experimental.pallas import tpu as pltpu
from jax.experimental.pallas import tpu_sc as plsc

_GAMMA = 12.0
_D = 64
_B = 16384
_NC = 2
_NS = 16
_NW = _NC * _NS
_BPW = _B // _NW
_CROWS = 16
_NCHUNK = _BPW // _CROWS


def _tec_body(hidx_hbm, tidx_hbm, ridx_hbm, ent_hbm, rel_hbm, out_hbm,
              hidx_v, tidx_v, ridx_v, h_v, t_v, r_v, out_v, sem):
    c = lax.axis_index("c")
    s = lax.axis_index("s")
    wid = s * _NC + c
    base = wid * _BPW

    pltpu.sync_copy(hidx_hbm.at[pl.ds(base, _BPW)], hidx_v)
    pltpu.sync_copy(tidx_hbm.at[pl.ds(base, _BPW)], tidx_v)
    pltpu.sync_copy(ridx_hbm.at[pl.ds(base, _BPW)], ridx_v)

    lane = lax.iota(jnp.int32, 16)

    def fire(ch, bufpar):
        rowbase = ch * _CROWS
        hiv = hidx_v[pl.ds(rowbase, 16)]
        tiv = tidx_v[pl.ds(rowbase, 16)]
        riv = ridx_v[pl.ds(rowbase, 16)]
        for i in range(16):
            he = hiv[i]
            te = tiv[i]
            pltpu.async_copy(ent_hbm.at[pl.ds(he >> 3, 1), he & 7, :],
                             h_v.at[bufpar, pl.ds(i, 1), :], sem)
            pltpu.async_copy(ent_hbm.at[pl.ds(te >> 3, 1), te & 7, :],
                             t_v.at[bufpar, pl.ds(i, 1), :], sem)
            pltpu.async_copy(rel_hbm.at[pl.ds(riv[i], 1), :],
                             r_v.at[bufpar, pl.ds(i, 1), :], sem)

    fire(0, 0)

    def chunk_body(ch, carry):
        bufpar = lax.rem(ch, 2)
        rowbase = ch * _CROWS
        pltpu.make_async_copy(rel_hbm.at[pl.ds(0, 16), :], h_v.at[0], sem).wait()
        pltpu.make_async_copy(rel_hbm.at[pl.ds(0, 16), :], t_v.at[0], sem).wait()
        pltpu.make_async_copy(rel_hbm.at[pl.ds(0, 16), :], r_v.at[0], sem).wait()

        @pl.when(ch + 1 < _NCHUNK)
        def _prefetch():
            fire(ch + 1, lax.rem(ch + 1, 2))

        vec = jnp.zeros((16,), jnp.float32)
        for i in range(16):
            acc = jnp.zeros((16,), jnp.float32)
            for cc in range(_D // 16):
                hv = h_v[bufpar, i, pl.ds(cc * 16, 16)]
                tv = t_v[bufpar, i, pl.ds(cc * 16, 16)]
                rv = r_v[bufpar, i, pl.ds(cc * 16, 16)]
                acc = acc + jnp.abs(hv + rv - tv)
            vec = jnp.where(lane == i, jnp.sum(acc), vec)
        out_v[pl.ds(rowbase, 16)] = _GAMMA - vec
        return carry

    lax.fori_loop(0, _NCHUNK, chunk_body, 0)

    pltpu.sync_copy(out_v, out_hbm.at[pl.ds(base, _BPW)])


@functools.partial(
    pl.kernel,
    out_type=jax.ShapeDtypeStruct((_B,), jnp.float32),
    mesh=plsc.VectorSubcoreMesh(core_axis_name="c", subcore_axis_name="s"),
    compiler_params=pltpu.CompilerParams(
        needs_layout_passes=False, use_tc_tiling_on_sc=True),
    scratch_types=[
        pltpu.VMEM((_BPW,), jnp.int32),
        pltpu.VMEM((_BPW,), jnp.int32),
        pltpu.VMEM((_BPW,), jnp.int32),
        pltpu.VMEM((2, _CROWS, _D), jnp.float32),
        pltpu.VMEM((2, _CROWS, _D), jnp.float32),
        pltpu.VMEM((2, _CROWS, _D), jnp.float32),
        pltpu.VMEM((_BPW,), jnp.float32),
        pltpu.SemaphoreType.DMA,
    ],
)
def _transe_sc(*refs):
    _tec_body(*refs)


def kernel(indices, relations, entity_embedding, relation_embedding):
    hidx = indices[:, 0].astype(jnp.int32)
    tidx = indices[:, 1].astype(jnp.int32)
    ridx = relations.astype(jnp.int32)
    ent3 = entity_embedding.reshape(entity_embedding.shape[0] // 8, 8, _D)
    return _transe_sc(hidx, tidx, ridx, ent3, relation_embedding)


# speedup vs baseline: 2.5473x; 1.0591x over previous
import functools

import jax
import jax.numpy as jnp
from jax import lax
from jax.experimental import pallas as pl
from jax.experimental.pallas import tpu as pltpu
from jax.experimental.pallas import tpu_sc as plsc

_GAMMA = 12.0
_D = 64
_B = 16384
_NC = 2
_NS = 16
_NW = _NC * _NS
_BPW = _B // _NW
_CROWS = 32
_NCHUNK = _BPW // _CROWS
_DEPTH = 3


def _tec_body(hidx_hbm, tidx_hbm, ridx_hbm, ent_hbm, rel_hbm, out_hbm,
              hidx_v, tidx_v, ridx_v, h_v, t_v, r_v, out_v, sem):
    c = lax.axis_index("c")
    s = lax.axis_index("s")
    wid = s * _NC + c
    base = wid * _BPW

    pltpu.sync_copy(hidx_hbm.at[pl.ds(base, _BPW)], hidx_v)
    pltpu.sync_copy(tidx_hbm.at[pl.ds(base, _BPW)], tidx_v)
    pltpu.sync_copy(ridx_hbm.at[pl.ds(base, _BPW)], ridx_v)

    lane = lax.iota(jnp.int32, 16)

    def fire(ch, bufpar):
        rowbase = ch * _CROWS
        for g in range(_CROWS // 16):
            hiv = hidx_v[pl.ds(rowbase + g * 16, 16)]
            tiv = tidx_v[pl.ds(rowbase + g * 16, 16)]
            riv = ridx_v[pl.ds(rowbase + g * 16, 16)]
            for i in range(16):
                r = g * 16 + i
                he = hiv[i]
                te = tiv[i]
                pltpu.async_copy(ent_hbm.at[pl.ds(he >> 3, 1), he & 7, :],
                                 h_v.at[bufpar, pl.ds(r, 1), :], sem)
                pltpu.async_copy(ent_hbm.at[pl.ds(te >> 3, 1), te & 7, :],
                                 t_v.at[bufpar, pl.ds(r, 1), :], sem)
                pltpu.async_copy(rel_hbm.at[pl.ds(riv[i], 1), :],
                                 r_v.at[bufpar, pl.ds(r, 1), :], sem)

    fire(0, 0)
    fire(1, 1)

    def chunk_body(ch, carry):
        bufpar = lax.rem(ch, _DEPTH)
        rowbase = ch * _CROWS
        pltpu.make_async_copy(rel_hbm.at[pl.ds(0, _CROWS), :], h_v.at[0], sem).wait()
        pltpu.make_async_copy(rel_hbm.at[pl.ds(0, _CROWS), :], t_v.at[0], sem).wait()
        pltpu.make_async_copy(rel_hbm.at[pl.ds(0, _CROWS), :], r_v.at[0], sem).wait()

        @pl.when(ch + 2 < _NCHUNK)
        def _prefetch():
            fire(ch + 2, lax.rem(ch + 2, _DEPTH))

        for g in range(_CROWS // 16):
            vec = jnp.zeros((16,), jnp.float32)
            for i in range(16):
                r = g * 16 + i
                acc = jnp.zeros((16,), jnp.float32)
                for cc in range(_D // 16):
                    hv = h_v[bufpar, r, pl.ds(cc * 16, 16)]
                    tv = t_v[bufpar, r, pl.ds(cc * 16, 16)]
                    rv = r_v[bufpar, r, pl.ds(cc * 16, 16)]
                    acc = acc + jnp.abs(hv + rv - tv)
                vec = jnp.where(lane == i, jnp.sum(acc), vec)
            out_v[pl.ds(rowbase + g * 16, 16)] = _GAMMA - vec
        return carry

    lax.fori_loop(0, _NCHUNK, chunk_body, 0)

    pltpu.sync_copy(out_v, out_hbm.at[pl.ds(base, _BPW)])


@functools.partial(
    pl.kernel,
    out_type=jax.ShapeDtypeStruct((_B,), jnp.float32),
    mesh=plsc.VectorSubcoreMesh(core_axis_name="c", subcore_axis_name="s"),
    compiler_params=pltpu.CompilerParams(
        needs_layout_passes=False, use_tc_tiling_on_sc=True),
    scratch_types=[
        pltpu.VMEM((_BPW,), jnp.int32),
        pltpu.VMEM((_BPW,), jnp.int32),
        pltpu.VMEM((_BPW,), jnp.int32),
        pltpu.VMEM((_DEPTH, _CROWS, _D), jnp.float32),
        pltpu.VMEM((_DEPTH, _CROWS, _D), jnp.float32),
        pltpu.VMEM((_DEPTH, _CROWS, _D), jnp.float32),
        pltpu.VMEM((_BPW,), jnp.float32),
        pltpu.SemaphoreType.DMA,
    ],
)
def _transe_sc(*refs):
    _tec_body(*refs)


def kernel(indices, relations, entity_embedding, relation_embedding):
    hidx = indices[:, 0].astype(jnp.int32)
    tidx = indices[:, 1].astype(jnp.int32)
    ridx = relations.astype(jnp.int32)
    ent3 = entity_embedding.reshape(entity_embedding.shape[0] // 8, 8, _D)
    return _transe_sc(hidx, tidx, ridx, ent3, relation_embedding)
